# baseline (speedup 1.0000x reference)
import functools

import jax
import jax.numpy as jnp
from jax import lax
from jax.experimental import pallas as pl
from jax.experimental.pallas import tpu as pltpu
from jax.experimental.pallas import tpu_sc as plsc

VOCAB = 100000
HIDDEN = 768
MAX_POS = 2048
BATCH = 4
SEQ = 2048
EPS = 1e-12

NC = 2
NS = 16
NW = NC * NS
TOKENS = BATCH * SEQ
TPW = TOKENS // NW
CHUNK = 64
NCHUNK = TPW // CHUNK
NSL = HIDDEN // 16


def _pp_body(pos_ref, tt_ref, o_ref):
    o_ref[...] = pos_ref[...] + tt_ref[...]


def _fold_type_row(position_embeddings, type_row):
    return pl.pallas_call(
        _pp_body,
        out_shape=jax.ShapeDtypeStruct((MAX_POS, HIDDEN), jnp.float32),
    )(position_embeddings, type_row.reshape(1, HIDDEN))


_GDN = lax.GatherDimensionNumbers(
    offset_dims=(), collapsed_slice_dims=(0,), start_index_map=(0,))


def _shuffle16(v, perm):
    return lax.gather(v, perm[:, None], _GDN, slice_sizes=(1,),
                      mode=lax.GatherScatterMode.PROMISE_IN_BOUNDS)


def _rsqrt16(x):
    i = lax.bitcast_convert_type(x, jnp.int32)
    i = jnp.int32(0x5F3759DF) - lax.shift_right_logical(i, 1)
    y = lax.bitcast_convert_type(i, jnp.float32)
    half = x * 0.5
    for _ in range(3):
        y = y * (1.5 - half * y * y)
    return y


def _sc_body(idx_hbm, word_hbm, pp_hbm, gam_hbm, bet_hbm, out_hbm,
             idx_v, rows_v, pos_v, gam_v, bet_v, gsem, psem):
    wid = lax.axis_index("s") * NC + lax.axis_index("c")
    tok_base = wid * TPW
    pos_base = lax.rem(tok_base, SEQ)

    pltpu.sync_copy(gam_hbm, gam_v)
    pltpu.sync_copy(bet_hbm, bet_v)
    pltpu.sync_copy(idx_hbm.at[wid], idx_v)

    def chunk_step(c, _):
        gcp = pltpu.async_copy(word_hbm.at[idx_v.at[c]], rows_v, gsem)
        pcp = pltpu.async_copy(
            pp_hbm.at[pl.ds(pos_base + c * CHUNK, CHUNK)], pos_v, psem)
        gcp.wait()
        pcp.wait()

        def token_step(t, _):
            acc_s = jnp.zeros((16,), jnp.float32)
            acc_q = jnp.zeros((16,), jnp.float32)
            xs = []
            for j in range(NSL):
                sl = pl.ds(j * 16, 16)
                v = rows_v[t, sl] + pos_v[t, sl]
                xs.append(v)
                acc_s = acc_s + v
                acc_q = acc_q + v * v
            lanes = lax.iota(jnp.int32, 16)
            for off in (8, 4, 2, 1):
                perm = lax.bitwise_xor(lanes, jnp.full((16,), off, jnp.int32))
                acc_s = acc_s + _shuffle16(acc_s, perm)
                acc_q = acc_q + _shuffle16(acc_q, perm)
            mean = acc_s * (1.0 / HIDDEN)
            var = acc_q * (1.0 / HIDDEN) - mean * mean
            rstd = _rsqrt16(var + EPS)
            mrs = mean * rstd
            for j in range(NSL):
                sl = pl.ds(j * 16, 16)
                y = (xs[j] * rstd - mrs) * gam_v[sl] + bet_v[sl]
                rows_v[t, sl] = y
            return ()

        lax.fori_loop(0, CHUNK, token_step, (), unroll=False)
        pltpu.sync_copy(
            rows_v, out_hbm.at[pl.ds(tok_base + c * CHUNK, CHUNK)])
        return ()

    lax.fori_loop(0, NCHUNK, chunk_step, (), unroll=False)


@jax.jit
def _bert_embed(idx3, word_embeddings, pp, ln_gamma, ln_beta):
    sc = pl.kernel(
        _sc_body,
        out_type=jax.ShapeDtypeStruct((TOKENS, HIDDEN), jnp.float32),
        mesh=plsc.VectorSubcoreMesh(
            core_axis_name="c", subcore_axis_name="s"),
        scratch_types=[
            pltpu.VMEM((NCHUNK, CHUNK), jnp.int32),
            pltpu.VMEM((CHUNK, HIDDEN), jnp.float32),
            pltpu.VMEM((CHUNK, HIDDEN), jnp.float32),
            pltpu.VMEM((HIDDEN,), jnp.float32),
            pltpu.VMEM((HIDDEN,), jnp.float32),
            pltpu.SemaphoreType.DMA,
            pltpu.SemaphoreType.DMA,
        ],
    )
    return sc(idx3, word_embeddings, pp, ln_gamma, ln_beta)


def kernel(input_ids, word_embeddings, position_embeddings,
           token_type_embeddings, ln_gamma, ln_beta):
    pp = _fold_type_row(position_embeddings, token_type_embeddings[0])
    idx3 = input_ids.astype(jnp.int32).reshape(NW, NCHUNK, CHUNK)
    out = _bert_embed(idx3, word_embeddings, pp, ln_gamma, ln_beta)
    return out.reshape(BATCH, SEQ, HIDDEN)


# speedup vs baseline: 1.8065x; 1.8065x over previous
import functools

import jax
import jax.numpy as jnp
from jax import lax
from jax.experimental import pallas as pl
from jax.experimental.pallas import tpu as pltpu
from jax.experimental.pallas import tpu_sc as plsc

VOCAB = 100000
HIDDEN = 768
MAX_POS = 2048
BATCH = 4
SEQ = 2048
EPS = 1e-12

NC = 2
NS = 16
NW = NC * NS
TOKENS = BATCH * SEQ

PIECES = 2
TP = TOKENS // PIECES
TPW = TP // NW
CHUNK = 64
NCHUNK = TPW // CHUNK

BLK = 512


def _sc_gather_body(idx_hbm, word_hbm, out_hbm,
                    idx_v, buf0, buf1, sem0, sem1):
    wid = lax.axis_index("s") * NC + lax.axis_index("c")
    base = wid * TPW
    pltpu.sync_copy(idx_hbm.at[wid], idx_v)

    bufs = (buf0, buf1)
    sems = (sem0, sem1)
    cps = [None] * NCHUNK
    cps[0] = pltpu.async_copy(word_hbm.at[idx_v.at[0]], bufs[0], sems[0])
    for c in range(NCHUNK):
        if c + 1 < NCHUNK:
            cps[c + 1] = pltpu.async_copy(
                word_hbm.at[idx_v.at[c + 1]], bufs[(c + 1) % 2],
                sems[(c + 1) % 2])
        cps[c].wait()
        pltpu.sync_copy(bufs[c % 2],
                        out_hbm.at[pl.ds(base + c * CHUNK, CHUNK)])


def _ln_body(g_ref, pos_ref, tt_ref, gam_ref, bet_ref, o_ref):
    x = g_ref[...] + pos_ref[...] + tt_ref[...]
    mean = jnp.mean(x, axis=1, keepdims=True)
    xc = x - mean
    var = jnp.mean(xc * xc, axis=1, keepdims=True)
    o_ref[...] = xc * lax.rsqrt(var + EPS) * gam_ref[...] + bet_ref[...]


_ln_call = pl.pallas_call(
    _ln_body,
    grid=(TP // BLK,),
    in_specs=[
        pl.BlockSpec((BLK, HIDDEN), lambda i: (i, 0)),
        pl.BlockSpec((BLK, HIDDEN), lambda i: (i % (SEQ // BLK), 0)),
        pl.BlockSpec((1, HIDDEN), lambda i: (0, 0)),
        pl.BlockSpec((1, HIDDEN), lambda i: (0, 0)),
        pl.BlockSpec((1, HIDDEN), lambda i: (0, 0)),
    ],
    out_specs=pl.BlockSpec((BLK, HIDDEN), lambda i: (i, 0)),
    out_shape=jax.ShapeDtypeStruct((TP, HIDDEN), jnp.float32),
)


@jax.jit
def _bert_embed(idx4, word_embeddings, position_embeddings, tt_row,
                gam2, bet2):
    sc_gather = pl.kernel(
        _sc_gather_body,
        out_type=jax.ShapeDtypeStruct((TP, HIDDEN), jnp.float32),
        mesh=plsc.VectorSubcoreMesh(
            core_axis_name="c", subcore_axis_name="s"),
        scratch_types=[
            pltpu.VMEM((NCHUNK, CHUNK), jnp.int32),
            pltpu.VMEM((CHUNK, HIDDEN), jnp.float32),
            pltpu.VMEM((CHUNK, HIDDEN), jnp.float32),
            pltpu.SemaphoreType.DMA,
            pltpu.SemaphoreType.DMA,
        ],
    )
    outs = []
    for p in range(PIECES):
        g = sc_gather(idx4[p], word_embeddings)
        outs.append(_ln_call(g, position_embeddings, tt_row, gam2, bet2))
    return jnp.concatenate(outs, axis=0)


def kernel(input_ids, word_embeddings, position_embeddings,
           token_type_embeddings, ln_gamma, ln_beta):
    idx4 = input_ids.astype(jnp.int32).reshape(PIECES, NW, NCHUNK, CHUNK)
    out = _bert_embed(
        idx4, word_embeddings, position_embeddings,
        token_type_embeddings[0].reshape(1, HIDDEN),
        ln_gamma.reshape(1, HIDDEN), ln_beta.reshape(1, HIDDEN))
    return out.reshape(BATCH, SEQ, HIDDEN)


# speedup vs baseline: 2.2761x; 1.2600x over previous
import functools

import jax
import jax.numpy as jnp
from jax import lax
from jax.experimental import pallas as pl
from jax.experimental.pallas import tpu as pltpu
from jax.experimental.pallas import tpu_sc as plsc

VOCAB = 100000
HIDDEN = 768
MAX_POS = 2048
BATCH = 4
SEQ = 2048
EPS = 1e-12

NC = 2
NS = 16
NW = NC * NS
TOKENS = BATCH * SEQ

PIECES = 4
BLK = SEQ // PIECES
TP = BATCH * BLK
TPW = TP // NW


def _sc_gather_body(idx_hbm, word_hbm, out_hbm, idx_v, buf, sem):
    wid = lax.axis_index("s") * NC + lax.axis_index("c")
    pltpu.sync_copy(idx_hbm.at[wid], idx_v)
    pltpu.async_copy(word_hbm.at[idx_v], buf, sem).wait()
    pltpu.sync_copy(buf, out_hbm.at[pl.ds(wid * TPW, TPW)])


def _ln_first_body(g_ref, pos_ref, tt_ref, gam_ref, bet_ref, o_ref):
    x = g_ref[...] + pos_ref[...] + tt_ref[...]
    mean = jnp.mean(x, axis=1, keepdims=True)
    xc = x - mean
    var = jnp.mean(xc * xc, axis=1, keepdims=True)
    o_ref[...] = xc * lax.rsqrt(var + EPS) * gam_ref[...] + bet_ref[...]


def _ln_chain_body(g_ref, pos_ref, tt_ref, gam_ref, bet_ref, prev_ref,
                   o_ref):
    del prev_ref
    _ln_first_body(g_ref, pos_ref, tt_ref, gam_ref, bet_ref, o_ref)


def _make_ln_call(p):
    specs = [
        pl.BlockSpec((BLK, HIDDEN), lambda b: (b, 0)),
        pl.BlockSpec((BLK, HIDDEN), lambda b: (p, 0)),
        pl.BlockSpec((1, HIDDEN), lambda b: (0, 0)),
        pl.BlockSpec((1, HIDDEN), lambda b: (0, 0)),
        pl.BlockSpec((1, HIDDEN), lambda b: (0, 0)),
    ]
    out_spec = pl.BlockSpec((BLK, HIDDEN), lambda b: (PIECES * b + p, 0))
    out_shape = jax.ShapeDtypeStruct((TOKENS, HIDDEN), jnp.float32)
    if p == 0:
        return pl.pallas_call(
            _ln_first_body, grid=(BATCH,), in_specs=specs,
            out_specs=out_spec, out_shape=out_shape)
    return pl.pallas_call(
        _ln_chain_body, grid=(BATCH,),
        in_specs=specs + [pl.BlockSpec(memory_space=pltpu.MemorySpace.HBM)],
        out_specs=out_spec, out_shape=out_shape,
        input_output_aliases={5: 0})


@jax.jit
def _bert_embed(idx3, word_embeddings, position_embeddings, tt_row,
                gam2, bet2):
    sc_gather = pl.kernel(
        _sc_gather_body,
        out_type=jax.ShapeDtypeStruct((TP, HIDDEN), jnp.float32),
        mesh=plsc.VectorSubcoreMesh(
            core_axis_name="c", subcore_axis_name="s"),
        scratch_types=[
            pltpu.VMEM((TPW,), jnp.int32),
            pltpu.VMEM((TPW, HIDDEN), jnp.float32),
            pltpu.SemaphoreType.DMA,
        ],
    )
    gs = [sc_gather(idx3[p], word_embeddings) for p in range(PIECES)]
    out = _make_ln_call(0)(gs[0], position_embeddings, tt_row, gam2, bet2)
    for p in range(1, PIECES):
        out = _make_ln_call(p)(
            gs[p], position_embeddings, tt_row, gam2, bet2, out)
    return out


def kernel(input_ids, word_embeddings, position_embeddings,
           token_type_embeddings, ln_gamma, ln_beta):
    idx3 = (input_ids.astype(jnp.int32)
            .reshape(BATCH, PIECES, BLK)
            .transpose(1, 0, 2)
            .reshape(PIECES, NW, TPW))
    out = _bert_embed(
        idx3, word_embeddings, position_embeddings,
        token_type_embeddings[0].reshape(1, HIDDEN),
        ln_gamma.reshape(1, HIDDEN), ln_beta.reshape(1, HIDDEN))
    return out.reshape(BATCH, SEQ, HIDDEN)


# speedup vs baseline: 2.2993x; 1.0102x over previous
import functools

import jax
import jax.numpy as jnp
from jax import lax
from jax.experimental import pallas as pl
from jax.experimental.pallas import tpu as pltpu
from jax.experimental.pallas import tpu_sc as plsc

VOCAB = 100000
HIDDEN = 768
MAX_POS = 2048
BATCH = 4
SEQ = 2048
EPS = 1e-12

NC = 2
NS = 16
NW = NC * NS
TOKENS = BATCH * SEQ

PIECES = 4
BLK = SEQ // PIECES
TP = BATCH * BLK
TPW = TP // NW


def _sc_gather_body(p, idx_hbm, word_hbm, out_hbm, idx_v, buf, sem):
    wid = lax.axis_index("s") * NC + lax.axis_index("c")
    base = (wid // 8) * SEQ + BLK * p + TPW * (wid % 8)
    pltpu.sync_copy(idx_hbm.at[pl.ds(base, TPW)], idx_v)
    pltpu.async_copy(word_hbm.at[idx_v], buf, sem).wait()
    pltpu.sync_copy(buf, out_hbm.at[pl.ds(wid * TPW, TPW)])


def _ln_first_body(g_ref, pos_ref, tt_ref, gam_ref, bet_ref, o_ref):
    x = g_ref[...] + pos_ref[...] + tt_ref[...]
    mean = jnp.mean(x, axis=1, keepdims=True)
    xc = x - mean
    var = jnp.mean(xc * xc, axis=1, keepdims=True)
    o_ref[...] = xc * lax.rsqrt(var + EPS) * gam_ref[...] + bet_ref[...]


def _ln_chain_body(g_ref, pos_ref, tt_ref, gam_ref, bet_ref, prev_ref,
                   o_ref):
    del prev_ref
    _ln_first_body(g_ref, pos_ref, tt_ref, gam_ref, bet_ref, o_ref)


def _make_ln_call(p):
    specs = [
        pl.BlockSpec((BLK, HIDDEN), lambda b: (b, 0)),
        pl.BlockSpec((BLK, HIDDEN), lambda b: (p, 0)),
        pl.BlockSpec((1, HIDDEN), lambda b: (0, 0)),
        pl.BlockSpec((1, HIDDEN), lambda b: (0, 0)),
        pl.BlockSpec((1, HIDDEN), lambda b: (0, 0)),
    ]
    out_spec = pl.BlockSpec((BLK, HIDDEN), lambda b: (PIECES * b + p, 0))
    out_shape = jax.ShapeDtypeStruct((TOKENS, HIDDEN), jnp.float32)
    if p == 0:
        return pl.pallas_call(
            _ln_first_body, grid=(BATCH,), in_specs=specs,
            out_specs=out_spec, out_shape=out_shape)
    return pl.pallas_call(
        _ln_chain_body, grid=(BATCH,),
        in_specs=specs + [pl.BlockSpec(memory_space=pltpu.MemorySpace.HBM)],
        out_specs=out_spec, out_shape=out_shape,
        input_output_aliases={5: 0})


@jax.jit
def _bert_embed(input_ids, word_embeddings, position_embeddings,
                token_type_embeddings, ln_gamma, ln_beta):
    idx_flat = input_ids.astype(jnp.int32).reshape(TOKENS)
    tt_row = token_type_embeddings[0].reshape(1, HIDDEN)
    gam2 = ln_gamma.reshape(1, HIDDEN)
    bet2 = ln_beta.reshape(1, HIDDEN)
    gs = []
    for p in range(PIECES):
        sc_gather = pl.kernel(
            functools.partial(_sc_gather_body, p),
            out_type=jax.ShapeDtypeStruct((TP, HIDDEN), jnp.float32),
            mesh=plsc.VectorSubcoreMesh(
                core_axis_name="c", subcore_axis_name="s"),
            scratch_types=[
                pltpu.VMEM((TPW,), jnp.int32),
                pltpu.VMEM((TPW, HIDDEN), jnp.float32),
                pltpu.SemaphoreType.DMA,
            ],
        )
        gs.append(sc_gather(idx_flat, word_embeddings))
    out = _make_ln_call(0)(gs[0], position_embeddings, tt_row, gam2, bet2)
    for p in range(1, PIECES):
        out = _make_ln_call(p)(
            gs[p], position_embeddings, tt_row, gam2, bet2, out)
    return out.reshape(BATCH, SEQ, HIDDEN)


def kernel(input_ids, word_embeddings, position_embeddings,
           token_type_embeddings, ln_gamma, ln_beta):
    return _bert_embed(input_ids, word_embeddings, position_embeddings,
                       token_type_embeddings, ln_gamma, ln_beta)


# speedup vs baseline: 2.5499x; 1.1090x over previous
import functools

import jax
import jax.numpy as jnp
from jax import lax
from jax.experimental import pallas as pl
from jax.experimental.pallas import tpu as pltpu
from jax.experimental.pallas import tpu_sc as plsc

VOCAB = 100000
HIDDEN = 768
MAX_POS = 2048
BATCH = 4
SEQ = 2048
EPS = 1e-12

NC = 2
NS = 16
NW = NC * NS
TOKENS = BATCH * SEQ

PIECES = 2
BLK = SEQ // PIECES
TP = BATCH * BLK
TPW = TP // NW


def _sc_gather_body(p, idx_hbm, word_hbm, out_hbm, idx_v, buf, sem):
    wid = lax.axis_index("s") * NC + lax.axis_index("c")
    base = (wid // 8) * SEQ + BLK * p + TPW * (wid % 8)
    pltpu.sync_copy(idx_hbm.at[pl.ds(base, TPW)], idx_v)
    pltpu.async_copy(word_hbm.at[idx_v], buf, sem).wait()
    pltpu.sync_copy(buf, out_hbm.at[pl.ds(wid * TPW, TPW)])


def _ln_first_body(g_ref, pos_ref, tt_ref, gam_ref, bet_ref, o_ref):
    x = g_ref[...] + pos_ref[...] + tt_ref[...]
    mean = jnp.mean(x, axis=1, keepdims=True)
    xc = x - mean
    var = jnp.mean(xc * xc, axis=1, keepdims=True)
    o_ref[...] = xc * lax.rsqrt(var + EPS) * gam_ref[...] + bet_ref[...]


def _ln_chain_body(g_ref, pos_ref, tt_ref, gam_ref, bet_ref, prev_ref,
                   o_ref):
    del prev_ref
    _ln_first_body(g_ref, pos_ref, tt_ref, gam_ref, bet_ref, o_ref)


def _make_ln_call(p):
    specs = [
        pl.BlockSpec((BLK, HIDDEN), lambda b: (b, 0)),
        pl.BlockSpec((BLK, HIDDEN), lambda b: (p, 0)),
        pl.BlockSpec((1, HIDDEN), lambda b: (0, 0)),
        pl.BlockSpec((1, HIDDEN), lambda b: (0, 0)),
        pl.BlockSpec((1, HIDDEN), lambda b: (0, 0)),
    ]
    out_spec = pl.BlockSpec((BLK, HIDDEN), lambda b: (PIECES * b + p, 0))
    out_shape = jax.ShapeDtypeStruct((TOKENS, HIDDEN), jnp.float32)
    if p == 0:
        return pl.pallas_call(
            _ln_first_body, grid=(BATCH,), in_specs=specs,
            out_specs=out_spec, out_shape=out_shape)
    return pl.pallas_call(
        _ln_chain_body, grid=(BATCH,),
        in_specs=specs + [pl.BlockSpec(memory_space=pltpu.MemorySpace.HBM)],
        out_specs=out_spec, out_shape=out_shape,
        input_output_aliases={5: 0})


@jax.jit
def _bert_embed(input_ids, word_embeddings, position_embeddings,
                token_type_embeddings, ln_gamma, ln_beta):
    idx_flat = input_ids.astype(jnp.int32).reshape(TOKENS)
    tt_row = token_type_embeddings[0].reshape(1, HIDDEN)
    gam2 = ln_gamma.reshape(1, HIDDEN)
    bet2 = ln_beta.reshape(1, HIDDEN)
    gs = []
    for p in range(PIECES):
        sc_gather = pl.kernel(
            functools.partial(_sc_gather_body, p),
            out_type=jax.ShapeDtypeStruct((TP, HIDDEN), jnp.float32),
            mesh=plsc.VectorSubcoreMesh(
                core_axis_name="c", subcore_axis_name="s"),
            scratch_types=[
                pltpu.VMEM((TPW,), jnp.int32),
                pltpu.VMEM((TPW, HIDDEN), jnp.float32),
                pltpu.SemaphoreType.DMA,
            ],
        )
        gs.append(sc_gather(idx_flat, word_embeddings))
    out = _make_ln_call(0)(gs[0], position_embeddings, tt_row, gam2, bet2)
    for p in range(1, PIECES):
        out = _make_ln_call(p)(
            gs[p], position_embeddings, tt_row, gam2, bet2, out)
    return out.reshape(BATCH, SEQ, HIDDEN)


def kernel(input_ids, word_embeddings, position_embeddings,
           token_type_embeddings, ln_gamma, ln_beta):
    return _bert_embed(input_ids, word_embeddings, position_embeddings,
                       token_type_embeddings, ln_gamma, ln_beta)


# speedup vs baseline: 2.5687x; 1.0074x over previous
import functools

import jax
import jax.numpy as jnp
from jax import lax
from jax.experimental import pallas as pl
from jax.experimental.pallas import tpu as pltpu
from jax.experimental.pallas import tpu_sc as plsc

VOCAB = 100000
HIDDEN = 768
MAX_POS = 2048
BATCH = 4
SEQ = 2048
EPS = 1e-12

NC = 2
NS = 16
NW = NC * NS
TOKENS = BATCH * SEQ

PIECES = 2
BLK = SEQ // PIECES
TP = BATCH * BLK
TPW = TP // NW


NCHUNK = 2
CW = TPW // NCHUNK


def _sc_gather_body(p, idx_hbm, word_hbm, out_hbm, idx_v, buf0, buf1, sem0,
                    sem1):
    wid = lax.axis_index("s") * NC + lax.axis_index("c")
    base = (wid // 8) * SEQ + BLK * p + TPW * (wid % 8)
    pltpu.sync_copy(idx_hbm.at[pl.ds(base, TPW)], idx_v)
    bufs = (buf0, buf1)
    sems = (sem0, sem1)
    cps = [None] * NCHUNK
    cps[0] = pltpu.async_copy(
        word_hbm.at[idx_v.at[pl.ds(0, CW)]], bufs[0], sems[0])
    for c in range(NCHUNK):
        if c + 1 < NCHUNK:
            cps[c + 1] = pltpu.async_copy(
                word_hbm.at[idx_v.at[pl.ds((c + 1) * CW, CW)]],
                bufs[(c + 1) % 2], sems[(c + 1) % 2])
        cps[c].wait()
        pltpu.sync_copy(bufs[c % 2],
                        out_hbm.at[pl.ds(wid * TPW + c * CW, CW)])


def _ln_first_body(g_ref, pos_ref, tt_ref, gam_ref, bet_ref, o_ref):
    x = g_ref[...] + pos_ref[...] + tt_ref[...]
    mean = jnp.mean(x, axis=1, keepdims=True)
    xc = x - mean
    var = jnp.mean(xc * xc, axis=1, keepdims=True)
    o_ref[...] = xc * lax.rsqrt(var + EPS) * gam_ref[...] + bet_ref[...]


def _ln_chain_body(g_ref, pos_ref, tt_ref, gam_ref, bet_ref, prev_ref,
                   o_ref):
    del prev_ref
    _ln_first_body(g_ref, pos_ref, tt_ref, gam_ref, bet_ref, o_ref)


def _make_ln_call(p):
    specs = [
        pl.BlockSpec((BLK, HIDDEN), lambda b: (b, 0)),
        pl.BlockSpec((BLK, HIDDEN), lambda b: (p, 0)),
        pl.BlockSpec((1, HIDDEN), lambda b: (0, 0)),
        pl.BlockSpec((1, HIDDEN), lambda b: (0, 0)),
        pl.BlockSpec((1, HIDDEN), lambda b: (0, 0)),
    ]
    out_spec = pl.BlockSpec((BLK, HIDDEN), lambda b: (PIECES * b + p, 0))
    out_shape = jax.ShapeDtypeStruct((TOKENS, HIDDEN), jnp.float32)
    if p == 0:
        return pl.pallas_call(
            _ln_first_body, grid=(BATCH,), in_specs=specs,
            out_specs=out_spec, out_shape=out_shape)
    return pl.pallas_call(
        _ln_chain_body, grid=(BATCH,),
        in_specs=specs + [pl.BlockSpec(memory_space=pltpu.MemorySpace.HBM)],
        out_specs=out_spec, out_shape=out_shape,
        input_output_aliases={5: 0})


@jax.jit
def _bert_embed(input_ids, word_embeddings, position_embeddings,
                token_type_embeddings, ln_gamma, ln_beta):
    idx_flat = input_ids.astype(jnp.int32).reshape(TOKENS)
    tt_row = token_type_embeddings[0].reshape(1, HIDDEN)
    gam2 = ln_gamma.reshape(1, HIDDEN)
    bet2 = ln_beta.reshape(1, HIDDEN)
    gs = []
    for p in range(PIECES):
        sc_gather = pl.kernel(
            functools.partial(_sc_gather_body, p),
            out_type=jax.ShapeDtypeStruct((TP, HIDDEN), jnp.float32),
            mesh=plsc.VectorSubcoreMesh(
                core_axis_name="c", subcore_axis_name="s"),
            scratch_types=[
                pltpu.VMEM((TPW,), jnp.int32),
                pltpu.VMEM((CW, HIDDEN), jnp.float32),
                pltpu.VMEM((CW, HIDDEN), jnp.float32),
                pltpu.SemaphoreType.DMA,
                pltpu.SemaphoreType.DMA,
            ],
        )
        gs.append(sc_gather(idx_flat, word_embeddings))
    out = _make_ln_call(0)(gs[0], position_embeddings, tt_row, gam2, bet2)
    for p in range(1, PIECES):
        out = _make_ln_call(p)(
            gs[p], position_embeddings, tt_row, gam2, bet2, out)
    return out.reshape(BATCH, SEQ, HIDDEN)


def kernel(input_ids, word_embeddings, position_embeddings,
           token_type_embeddings, ln_gamma, ln_beta):
    return _bert_embed(input_ids, word_embeddings, position_embeddings,
                       token_type_embeddings, ln_gamma, ln_beta)
